# baseline (speedup 1.0000x reference)
import functools

import numpy as np

import jax
import jax.numpy as jnp
from jax import lax
from jax.experimental import pallas as pl
from jax.experimental.pallas import tpu as pltpu
from jax.experimental.pallas import tpu_sc as plsc

N_NODES = 10000
N_PAD = 10240
N_EDGES = 320000
D = 128
DEGW = 16

NC = 2
NS = 16
NW = NC * NS
CHUNK = 128
NCHUNK = 80
EPW = NCHUNK * CHUNK
E_PAD = NW * EPW
CPP = 8
NPHASE = NCHUNK // CPP
RPS = N_PAD // NS

_NTAIL = NW * NCHUNK * CHUNK - N_EDGES
_SRC_TAIL = jnp.asarray(np.arange(_NTAIL) % N_NODES, dtype=jnp.int32)
_DST_TAIL = jnp.asarray(
    N_NODES + np.arange(_NTAIL) % (N_PAD - N_NODES), dtype=jnp.int32)


def _sc_aggregate(x, src3d, dst3d):
    mesh = plsc.VectorSubcoreMesh(core_axis_name="c", subcore_axis_name="s")

    @functools.partial(
        pl.kernel,
        mesh=mesh,
        out_type=jax.ShapeDtypeStruct((NC, N_PAD, D), jnp.float32),
        scratch_types=[
            pltpu.VMEM((CPP, CHUNK), jnp.int32),
            pltpu.VMEM((CPP, CHUNK), jnp.int32),
            pltpu.VMEM((CPP, CHUNK), jnp.int32),
            pltpu.VMEM((CPP, CHUNK), jnp.int32),
            pltpu.VMEM((CHUNK, D), jnp.float32),
            pltpu.VMEM((CHUNK, D), jnp.float32),
            pltpu.VMEM_SHARED((N_PAD, D), jnp.float32),
            pltpu.SemaphoreType.DMA,
            pltpu.SemaphoreType.DMA,
            pltpu.SemaphoreType.DMA,
            pltpu.SemaphoreType.DMA,
            pltpu.SemaphoreType.DMA,
        ],
        compiler_params=pltpu.CompilerParams(use_tc_tiling_on_sc=True),
    )
    def body(x_hbm, src_hbm, dst_hbm, agg_out, src_v0, dst_v0, src_v1,
             dst_v1, rows_a, rows_b, agg_sh, sem_a, sem_b, sem_i,
             ssa, ssb):
        c = lax.axis_index("c")
        s = lax.axis_index("s")
        wid = c * NS + s
        slabs = ((src_v0, dst_v0), (src_v1, dst_v1))

        pltpu.async_copy(src_hbm.at[wid, pl.ds(0, CPP)], src_v0, sem_i)
        pltpu.async_copy(dst_hbm.at[wid, pl.ds(0, CPP)], dst_v0, sem_i)

        def zero_rows(i, carry):
            for k in range(D // 16):
                rows_a[i, pl.ds(k * 16, 16)] = jnp.zeros((16,), jnp.float32)
                rows_b[i, pl.ds(k * 16, 16)] = jnp.zeros((16,), jnp.float32)
            return carry
        lax.fori_loop(0, CHUNK, zero_rows, 0)
        for b in range(RPS // CHUNK):
            pltpu.sync_copy(rows_a,
                            agg_sh.at[pl.ds(s * RPS + b * CHUNK, CHUNK)])

        def iwait(sv, dv):
            pltpu.make_async_copy(src_hbm.at[wid, pl.ds(0, CPP)], sv,
                                  sem_i).wait()
            pltpu.make_async_copy(dst_hbm.at[wid, pl.ds(0, CPP)], dv,
                                  sem_i).wait()

        def gwait(buf, sem):
            pltpu.make_async_copy(x_hbm.at[pl.ds(0, CHUNK)], buf, sem).wait()

        def swait(buf, sem):
            pltpu.make_async_copy(buf, agg_sh.at[pl.ds(0, CHUNK)],
                                  sem).wait()

        iwait(src_v0, dst_v0)
        pltpu.async_copy(x_hbm.at[src_v0.at[0]], rows_a, sem_a)

        plsc.subcore_barrier()

        pltpu.async_copy(rows_b, agg_sh.at[dst_v0.at[0]], ssa, add=True)
        pltpu.async_copy(rows_b, agg_sh.at[dst_v0.at[0]], ssb, add=True)

        for p in range(NPHASE):
            sv, dv = slabs[p % 2]
            nsv, ndv = slabs[(p + 1) % 2]
            if p + 1 < NPHASE:
                pltpu.async_copy(
                    src_hbm.at[wid, pl.ds((p + 1) * CPP, CPP)], nsv, sem_i)
                pltpu.async_copy(
                    dst_hbm.at[wid, pl.ds((p + 1) * CPP, CPP)], ndv, sem_i)

            def pair(g, carry2, sv=sv, dv=dv):
                j0 = 2 * g
                gwait(rows_a, sem_a)
                swait(rows_b, ssb)
                pltpu.async_copy(x_hbm.at[sv.at[j0 + 1]], rows_b, sem_b)
                pltpu.async_copy(rows_a, agg_sh.at[dv.at[j0]], ssa,
                                 add=True)
                gwait(rows_b, sem_b)
                swait(rows_a, ssa)
                pltpu.async_copy(x_hbm.at[sv.at[j0 + 2]], rows_a, sem_a)
                pltpu.async_copy(rows_b, agg_sh.at[dv.at[j0 + 1]], ssb,
                                 add=True)
                return carry2
            lax.fori_loop(0, CPP // 2 - 1, pair, 0)

            j0 = CPP - 2
            gwait(rows_a, sem_a)
            swait(rows_b, ssb)
            pltpu.async_copy(x_hbm.at[sv.at[j0 + 1]], rows_b, sem_b)
            pltpu.async_copy(rows_a, agg_sh.at[dv.at[j0]], ssa, add=True)
            gwait(rows_b, sem_b)
            swait(rows_a, ssa)
            if p + 1 < NPHASE:
                iwait(nsv, ndv)
                pltpu.async_copy(x_hbm.at[nsv.at[0]], rows_a, sem_a)
            pltpu.async_copy(rows_b, agg_sh.at[dv.at[j0 + 1]], ssb,
                             add=True)

        swait(rows_a, ssa)
        swait(rows_b, ssb)
        plsc.subcore_barrier()

        pltpu.sync_copy(agg_sh.at[pl.ds(s * RPS, RPS)],
                        agg_out.at[c, pl.ds(s * RPS, RPS)])

    return body(x, src3d, dst3d)


def _sc_degrees(dst3d):
    mesh = plsc.VectorSubcoreMesh(core_axis_name="c", subcore_axis_name="s")

    @functools.partial(
        pl.kernel,
        mesh=mesh,
        out_type=jax.ShapeDtypeStruct((NC, N_PAD, DEGW), jnp.float32),
        scratch_types=[
            pltpu.VMEM((NCHUNK, CHUNK), jnp.int32),
            pltpu.VMEM((CHUNK, DEGW), jnp.float32),
            pltpu.VMEM((CHUNK, DEGW), jnp.float32),
            pltpu.VMEM_SHARED((N_PAD, DEGW), jnp.float32),
        ],
        compiler_params=pltpu.CompilerParams(use_tc_tiling_on_sc=False),
    )
    def body(dst_hbm, deg_out, dst_v, ones_v, zer_v, deg_sh):
        c = lax.axis_index("c")
        s = lax.axis_index("s")
        wid = c * NS + s

        def fill(i, carry):
            ones_v[i, pl.ds(0, 16)] = jnp.ones((16,), jnp.float32)
            zer_v[i, pl.ds(0, 16)] = jnp.zeros((16,), jnp.float32)
            return carry
        lax.fori_loop(0, CHUNK, fill, 0)
        for b in range(RPS // CHUNK):
            pltpu.sync_copy(zer_v,
                            deg_sh.at[pl.ds(s * RPS + b * CHUNK, CHUNK)])

        pltpu.sync_copy(dst_hbm.at[wid], dst_v)

        plsc.subcore_barrier()

        def chunk(j, carry):
            pltpu.sync_copy(ones_v, deg_sh.at[dst_v.at[j]], add=True)
            return carry
        lax.fori_loop(0, NCHUNK, chunk, 0)

        plsc.subcore_barrier()

        pltpu.sync_copy(deg_sh.at[pl.ds(s * RPS, RPS)],
                        deg_out.at[c, pl.ds(s * RPS, RPS)])

    return body(dst3d)


def _tc_body(agg_ref, x_ref, deg_ref, w_ref, b_ref, out_ref):
    aggs = agg_ref[0] + agg_ref[1] + x_ref[...]
    deg = deg_ref[0, :, 0:1] + deg_ref[1, :, 0:1] + 1.0
    h = aggs / deg
    out_ref[...] = (
        jnp.dot(h, w_ref[...], preferred_element_type=jnp.float32) + b_ref[...]
    )


def _tc_finish(agg_part, x, deg_part, w, bias2d):
    blk = 2000
    grid = (N_NODES // blk,)
    return pl.pallas_call(
        _tc_body,
        grid=grid,
        in_specs=[
            pl.BlockSpec((NC, blk, D), lambda i: (0, i, 0)),
            pl.BlockSpec((blk, D), lambda i: (i, 0)),
            pl.BlockSpec((NC, blk, DEGW), lambda i: (0, i, 0)),
            pl.BlockSpec((D, D), lambda i: (0, 0)),
            pl.BlockSpec((1, D), lambda i: (0, 0)),
        ],
        out_specs=pl.BlockSpec((blk, D), lambda i: (i, 0)),
        out_shape=jax.ShapeDtypeStruct((N_NODES, D), jnp.float32),
    )(agg_part, x, deg_part, w, bias2d)


def kernel(x, edge_index, W_neigh, bias):
    src3d = jnp.concatenate(
        [edge_index[0], _SRC_TAIL]).reshape(NW, NCHUNK, CHUNK)
    dst3d = jnp.concatenate(
        [edge_index[1], _DST_TAIL]).reshape(NW, NCHUNK, CHUNK)
    agg_part = _sc_aggregate(x, src3d, dst3d)
    deg_part = _sc_degrees(dst3d)
    return _tc_finish(agg_part, x, deg_part, W_neigh, bias.reshape(1, D))


# speedup vs baseline: 1.0095x; 1.0095x over previous
import functools

import numpy as np

import jax
import jax.numpy as jnp
from jax import lax
from jax.experimental import pallas as pl
from jax.experimental.pallas import tpu as pltpu
from jax.experimental.pallas import tpu_sc as plsc

N_NODES = 10000
N_PAD = 10240
N_EDGES = 320000
D = 128
DEGW = 16

NC = 2
NS = 16
NW = NC * NS
CHUNK = 128
NCHUNK = 80
EPW = NCHUNK * CHUNK
E_PAD = NW * EPW
CPP = 8
NPHASE = NCHUNK // CPP
RPS = N_PAD // NS

_NTAIL = NW * NCHUNK * CHUNK - N_EDGES
_SRC_TAIL = jnp.asarray(np.arange(_NTAIL) % N_NODES, dtype=jnp.int32)
_DST_TAIL = jnp.asarray(
    N_NODES + np.arange(_NTAIL) % (N_PAD - N_NODES), dtype=jnp.int32)


def _sc_aggregate(x, src3d, dst3d):
    mesh = plsc.VectorSubcoreMesh(core_axis_name="c", subcore_axis_name="s")

    @functools.partial(
        pl.kernel,
        mesh=mesh,
        out_type=jax.ShapeDtypeStruct((NC, N_PAD, D), jnp.float32),
        scratch_types=[
            pltpu.VMEM((CPP, CHUNK), jnp.int32),
            pltpu.VMEM((CPP, CHUNK), jnp.int32),
            pltpu.VMEM((CPP, CHUNK), jnp.int32),
            pltpu.VMEM((CPP, CHUNK), jnp.int32),
            pltpu.VMEM((CHUNK, D), jnp.float32),
            pltpu.VMEM((CHUNK, D), jnp.float32),
            pltpu.VMEM_SHARED((N_PAD, D), jnp.float32),
            pltpu.SemaphoreType.DMA,
            pltpu.SemaphoreType.DMA,
            pltpu.SemaphoreType.DMA,
        ],
        compiler_params=pltpu.CompilerParams(use_tc_tiling_on_sc=True),
    )
    def body(x_hbm, src_hbm, dst_hbm, agg_out, src_v0, dst_v0, src_v1,
             dst_v1, rows_a, rows_b, agg_sh, sem_a, sem_b, sem_i):
        c = lax.axis_index("c")
        s = lax.axis_index("s")
        wid = c * NS + s
        slabs = ((src_v0, dst_v0), (src_v1, dst_v1))

        pltpu.async_copy(src_hbm.at[wid, pl.ds(0, CPP)], src_v0, sem_i)
        pltpu.async_copy(dst_hbm.at[wid, pl.ds(0, CPP)], dst_v0, sem_i)

        def zero_rows(i, carry):
            for k in range(D // 16):
                rows_a[i, pl.ds(k * 16, 16)] = jnp.zeros((16,), jnp.float32)
            return carry
        lax.fori_loop(0, CHUNK, zero_rows, 0)
        for b in range(RPS // CHUNK):
            pltpu.sync_copy(rows_a,
                            agg_sh.at[pl.ds(s * RPS + b * CHUNK, CHUNK)])

        def iwait(sv, dv):
            pltpu.make_async_copy(src_hbm.at[wid, pl.ds(0, CPP)], sv,
                                  sem_i).wait()
            pltpu.make_async_copy(dst_hbm.at[wid, pl.ds(0, CPP)], dv,
                                  sem_i).wait()

        def gwait(buf, sem):
            pltpu.make_async_copy(x_hbm.at[pl.ds(0, CHUNK)], buf, sem).wait()

        iwait(src_v0, dst_v0)
        pltpu.async_copy(x_hbm.at[src_v0.at[0]], rows_a, sem_a)

        plsc.subcore_barrier()

        for p in range(NPHASE):
            sv, dv = slabs[p % 2]
            nsv, ndv = slabs[(p + 1) % 2]
            if p + 1 < NPHASE:
                pltpu.async_copy(
                    src_hbm.at[wid, pl.ds((p + 1) * CPP, CPP)], nsv, sem_i)
                pltpu.async_copy(
                    dst_hbm.at[wid, pl.ds((p + 1) * CPP, CPP)], ndv, sem_i)

            def pair(g, carry2, sv=sv, dv=dv):
                j0 = 2 * g
                gwait(rows_a, sem_a)
                pltpu.async_copy(x_hbm.at[sv.at[j0 + 1]], rows_b, sem_b)
                pltpu.sync_copy(rows_a, agg_sh.at[dv.at[j0]], add=True)
                gwait(rows_b, sem_b)
                pltpu.async_copy(x_hbm.at[sv.at[j0 + 2]], rows_a, sem_a)
                pltpu.sync_copy(rows_b, agg_sh.at[dv.at[j0 + 1]], add=True)
                return carry2
            lax.fori_loop(0, CPP // 2 - 1, pair, 0)

            j0 = CPP - 2
            gwait(rows_a, sem_a)
            pltpu.async_copy(x_hbm.at[sv.at[j0 + 1]], rows_b, sem_b)
            pltpu.sync_copy(rows_a, agg_sh.at[dv.at[j0]], add=True)
            gwait(rows_b, sem_b)
            if p + 1 < NPHASE:
                iwait(nsv, ndv)
                pltpu.async_copy(x_hbm.at[nsv.at[0]], rows_a, sem_a)
            pltpu.sync_copy(rows_b, agg_sh.at[dv.at[j0 + 1]], add=True)

        plsc.subcore_barrier()

        pltpu.sync_copy(agg_sh.at[pl.ds(s * RPS, RPS)],
                        agg_out.at[c, pl.ds(s * RPS, RPS)])

    return body(x, src3d, dst3d)


def _sc_degrees(dst3d):
    mesh = plsc.VectorSubcoreMesh(core_axis_name="c", subcore_axis_name="s")

    @functools.partial(
        pl.kernel,
        mesh=mesh,
        out_type=jax.ShapeDtypeStruct((NC, N_PAD, DEGW), jnp.float32),
        scratch_types=[
            pltpu.VMEM((NCHUNK, CHUNK), jnp.int32),
            pltpu.VMEM((CHUNK, DEGW), jnp.float32),
            pltpu.VMEM((CHUNK, DEGW), jnp.float32),
            pltpu.VMEM_SHARED((N_PAD, DEGW), jnp.float32),
        ],
        compiler_params=pltpu.CompilerParams(use_tc_tiling_on_sc=False),
    )
    def body(dst_hbm, deg_out, dst_v, ones_v, zer_v, deg_sh):
        c = lax.axis_index("c")
        s = lax.axis_index("s")
        wid = c * NS + s

        def fill(i, carry):
            ones_v[i, pl.ds(0, 16)] = jnp.ones((16,), jnp.float32)
            zer_v[i, pl.ds(0, 16)] = jnp.zeros((16,), jnp.float32)
            return carry
        lax.fori_loop(0, CHUNK, fill, 0)
        for b in range(RPS // CHUNK):
            pltpu.sync_copy(zer_v,
                            deg_sh.at[pl.ds(s * RPS + b * CHUNK, CHUNK)])

        pltpu.sync_copy(dst_hbm.at[wid], dst_v)

        plsc.subcore_barrier()

        def chunk(j, carry):
            pltpu.sync_copy(ones_v, deg_sh.at[dst_v.at[j]], add=True)
            return carry
        lax.fori_loop(0, NCHUNK, chunk, 0)

        plsc.subcore_barrier()

        pltpu.sync_copy(deg_sh.at[pl.ds(s * RPS, RPS)],
                        deg_out.at[c, pl.ds(s * RPS, RPS)])

    return body(dst3d)


def _tc_body(agg_ref, x_ref, deg_ref, w_ref, b_ref, out_ref):
    aggs = agg_ref[0] + agg_ref[1] + x_ref[...]
    deg = deg_ref[0, :, 0:1] + deg_ref[1, :, 0:1] + 1.0
    h = aggs / deg
    out_ref[...] = (
        jnp.dot(h, w_ref[...], preferred_element_type=jnp.float32) + b_ref[...]
    )


def _tc_finish(agg_part, x, deg_part, w, bias2d):
    blk = 2000
    grid = (N_NODES // blk,)
    return pl.pallas_call(
        _tc_body,
        grid=grid,
        in_specs=[
            pl.BlockSpec((NC, blk, D), lambda i: (0, i, 0)),
            pl.BlockSpec((blk, D), lambda i: (i, 0)),
            pl.BlockSpec((NC, blk, DEGW), lambda i: (0, i, 0)),
            pl.BlockSpec((D, D), lambda i: (0, 0)),
            pl.BlockSpec((1, D), lambda i: (0, 0)),
        ],
        out_specs=pl.BlockSpec((blk, D), lambda i: (i, 0)),
        out_shape=jax.ShapeDtypeStruct((N_NODES, D), jnp.float32),
    )(agg_part, x, deg_part, w, bias2d)


def kernel(x, edge_index, W_neigh, bias):
    src3d = jnp.concatenate(
        [edge_index[0], _SRC_TAIL]).reshape(NW, NCHUNK, CHUNK)
    dst3d = jnp.concatenate(
        [edge_index[1], _DST_TAIL]).reshape(NW, NCHUNK, CHUNK)
    agg_part = _sc_aggregate(x, src3d, dst3d)
    deg_part = _sc_degrees(dst3d)
    return _tc_finish(agg_part, x, deg_part, W_neigh, bias.reshape(1, D))
